# baseline (speedup 1.0000x reference)
import functools

import jax
import jax.numpy as jnp
from jax import lax
from jax.experimental import pallas as pl
from jax.experimental.pallas import tpu as pltpu
from jax.experimental.pallas import tpu_sc as plsc

B = 16384
NS, NG, NF = 100000, 10000, 1000
DS, DT = 32, 8
KS, KT = 10, 3
DOUT = KS * DS + KT * DT + KT * DT

NC, NSUB = 2, 16
NW = NC * NSUB
BPW = B // NW
L = 16
GRP = BPW // L

_SLOTS = (
    [(i * DS, DS, NS, i) for i in range(KS)]
    + [(KS * DS + j * DT, DT, NG, j) for j in range(KT)]
    + [(KS * DS + KT * DT + j * DT, DT, NF, j) for j in range(KT)]
)


def _sc_body(sids, gids, fids, stab, gtab, ftab, out,
             sraw, graw, fraw, idxv, buf0, buf1, tb0, tb1, sem0, sem1):
    wid = lax.axis_index("s") * NC + lax.axis_index("c")
    base = wid * BPW
    iota = lax.iota(jnp.int32, L)

    pltpu.sync_copy(sids.at[pl.ds(base, BPW)], sraw)
    pltpu.sync_copy(gids.at[pl.ds(base, BPW)], graw)
    pltpu.sync_copy(fids.at[pl.ds(base, BPW)], fraw)

    zcounts = []
    for s, (_, _, nrows, sl) in enumerate(_SLOTS):
        raw = sraw if s < KS else (graw if s < KS + KT else fraw)
        off = sl * nrows
        colv = jnp.full((L,), sl, jnp.int32)

        def gbody(g, zc, raw=raw, off=off, colv=colv):
            rows = g * L + iota
            v = plsc.load_gather(raw, [rows, colv])
            zc = zc + plsc.all_reduce_population_count(v == 0)
            idxv[s, pl.ds(g * L, L)] = v + off
            return zc

        zc = lax.fori_loop(0, GRP, gbody, jnp.zeros((L,), jnp.int32))
        zcounts.append(jnp.max(zc))

    def table_for(s):
        return stab if s < KS else (gtab if s < KS + KT else ftab)

    def buf_for(s):
        big = _SLOTS[s][1] == DS
        pair = (buf0, buf1) if big else (tb0, tb1)
        return pair[s % 2]

    sems = (sem0, sem1)

    def start(s):
        return pltpu.async_copy(
            table_for(s).at[idxv.at[s]], buf_for(s), sems[s % 2])

    cp = start(0)
    for s, (col, d, nrows, sl) in enumerate(_SLOTS):
        cp.wait()
        if s + 1 < len(_SLOTS):
            cp = start(s + 1)
        buf = buf_for(s)
        off = sl * nrows
        zeros = jnp.zeros((L,), jnp.float32)

        @pl.when(zcounts[s] > 0)
        def _(buf=buf, off=off, s=s, d=d, zeros=zeros):
            def gbody(g, _):
                rows = g * L + iota
                m = idxv[s, pl.ds(g * L, L)] == off

                def cbody(c, _):
                    cv = jnp.zeros((L,), jnp.int32) + c
                    plsc.store_scatter(buf, [rows, cv], zeros, mask=m)
                    return 0

                lax.fori_loop(0, d, cbody, 0)
                return 0

            lax.fori_loop(0, GRP, gbody, 0)

        pltpu.sync_copy(buf, out.at[pl.ds(base, BPW), pl.ds(col, d)])


@jax.jit
def _run(sids, gids, fids, stab, gtab, ftab):
    mesh = plsc.VectorSubcoreMesh(core_axis_name="c", subcore_axis_name="s")
    return pl.kernel(
        _sc_body,
        out_type=jax.ShapeDtypeStruct((B, DOUT), jnp.float32),
        mesh=mesh,
        compiler_params=pltpu.CompilerParams(
            use_tc_tiling_on_sc=False, needs_layout_passes=False),
        scratch_types=[
            pltpu.VMEM((BPW, KS), jnp.int32),
            pltpu.VMEM((BPW, KT), jnp.int32),
            pltpu.VMEM((BPW, KT), jnp.int32),
            pltpu.VMEM((len(_SLOTS), BPW), jnp.int32),
            pltpu.VMEM((BPW, DS), jnp.float32),
            pltpu.VMEM((BPW, DS), jnp.float32),
            pltpu.VMEM((BPW, DT), jnp.float32),
            pltpu.VMEM((BPW, DT), jnp.float32),
            pltpu.SemaphoreType.DMA,
            pltpu.SemaphoreType.DMA,
        ],
    )(sids, gids, fids, stab, gtab, ftab)


def kernel(species_ids, genus_ids, family_ids, species_tables, genus_tables,
           family_tables):
    return _run(
        species_ids.astype(jnp.int32),
        genus_ids.astype(jnp.int32),
        family_ids.astype(jnp.int32),
        species_tables.reshape(KS * NS, DS),
        genus_tables.reshape(KT * NG, DT),
        family_tables.reshape(KT * NF, DT),
    )


# speedup vs baseline: 1.0206x; 1.0206x over previous
import functools

import jax
import jax.numpy as jnp
from jax import lax
from jax.experimental import pallas as pl
from jax.experimental.pallas import tpu as pltpu
from jax.experimental.pallas import tpu_sc as plsc

B = 16384
NS, NG, NF = 100000, 10000, 1000
DS, DT = 32, 8
KS, KT = 10, 3
DOUT = KS * DS + KT * DT + KT * DT

NC, NSUB = 2, 16
NW = NC * NSUB
BPW = B // NW
L = 16
GRP = BPW // L

_SLOTS = (
    [(i * DS, DS, NS, i) for i in range(KS)]
    + [(KS * DS + j * DT, DT, NG, j) for j in range(KT)]
    + [(KS * DS + KT * DT + j * DT, DT, NF, j) for j in range(KT)]
)


def _sc_body(sids, gids, fids, stab, gtab, ftab, out,
             sraw, graw, fraw, idxv, buf0, buf1, tb0, tb1, sem0, sem1):
    wid = lax.axis_index("s") * NC + lax.axis_index("c")
    base = wid * BPW
    iota = lax.iota(jnp.int32, L)

    pltpu.sync_copy(sids.at[pl.ds(base * KS, BPW * KS)], sraw)
    pltpu.sync_copy(gids.at[pl.ds(base * KT, BPW * KT)], graw)
    pltpu.sync_copy(fids.at[pl.ds(base * KT, BPW * KT)], fraw)

    zcounts = []
    for s, (_, _, nrows, sl) in enumerate(_SLOTS):
        raw, k = (sraw, KS) if s < KS else (
            (graw, KT) if s < KS + KT else (fraw, KT))
        off = sl * nrows

        def gbody(g, zc, raw=raw, off=off, sl=sl, k=k):
            rows = g * L + iota
            v = plsc.load_gather(raw, [rows * k + sl])
            zc = zc + plsc.all_reduce_population_count(v == 0)
            idxv[s, pl.ds(g * L, L)] = v + off
            return zc

        zc = lax.fori_loop(0, GRP, gbody, jnp.zeros((L,), jnp.int32))
        zcounts.append(jnp.max(zc))

    def table_for(s):
        return stab if s < KS else (gtab if s < KS + KT else ftab)

    def buf_for(s):
        big = _SLOTS[s][1] == DS
        pair = (buf0, buf1) if big else (tb0, tb1)
        return pair[s % 2]

    sems = (sem0, sem1)

    def start(s):
        return pltpu.async_copy(
            table_for(s).at[idxv.at[s]], buf_for(s), sems[s % 2])

    cp = start(0)
    for s, (col, d, nrows, sl) in enumerate(_SLOTS):
        cp.wait()
        if s + 1 < len(_SLOTS):
            cp = start(s + 1)
        buf = buf_for(s)
        off = sl * nrows
        zeros = jnp.zeros((L,), jnp.float32)

        @pl.when(zcounts[s] > 0)
        def _(buf=buf, off=off, s=s, d=d, zeros=zeros):
            def gbody(g, _):
                rows = g * L + iota
                m = idxv[s, pl.ds(g * L, L)] == off

                def cbody(c, _):
                    cv = jnp.zeros((L,), jnp.int32) + c
                    plsc.store_scatter(buf, [rows, cv], zeros, mask=m)
                    return 0

                lax.fori_loop(0, d, cbody, 0)
                return 0

            lax.fori_loop(0, GRP, gbody, 0)

        pltpu.sync_copy(buf, out.at[pl.ds(base, BPW), pl.ds(col, d)])


@jax.jit
def _run(sids, gids, fids, stab, gtab, ftab):
    mesh = plsc.VectorSubcoreMesh(core_axis_name="c", subcore_axis_name="s")
    return pl.kernel(
        _sc_body,
        out_type=jax.ShapeDtypeStruct((B, DOUT), jnp.float32),
        mesh=mesh,
        compiler_params=pltpu.CompilerParams(
            use_tc_tiling_on_sc=False, needs_layout_passes=False),
        scratch_types=[
            pltpu.VMEM((BPW * KS,), jnp.int32),
            pltpu.VMEM((BPW * KT,), jnp.int32),
            pltpu.VMEM((BPW * KT,), jnp.int32),
            pltpu.VMEM((len(_SLOTS), BPW), jnp.int32),
            pltpu.VMEM((BPW, DS), jnp.float32),
            pltpu.VMEM((BPW, DS), jnp.float32),
            pltpu.VMEM((BPW, DT), jnp.float32),
            pltpu.VMEM((BPW, DT), jnp.float32),
            pltpu.SemaphoreType.DMA,
            pltpu.SemaphoreType.DMA,
        ],
    )(sids, gids, fids, stab, gtab, ftab)


def kernel(species_ids, genus_ids, family_ids, species_tables, genus_tables,
           family_tables):
    return _run(
        species_ids.reshape(-1).astype(jnp.int32),
        genus_ids.reshape(-1).astype(jnp.int32),
        family_ids.reshape(-1).astype(jnp.int32),
        species_tables.reshape(KS * NS, DS),
        genus_tables.reshape(KT * NG, DT),
        family_tables.reshape(KT * NF, DT),
    )
